# Initial kernel scaffold; baseline (speedup 1.0000x reference)
import jax
import jax.numpy as jnp
from jax.experimental import pallas as pl


def kernel(x, edge_index, W1, b1, W2, b2, temp):
    raise NotImplementedError("write your pallas kernel here")


# speedup vs baseline: 508.4453x; 508.4453x over previous
import functools

import numpy as np

import jax
import jax.numpy as jnp
from jax import lax
from jax.experimental import pallas as pl
from jax.experimental.pallas import tpu as pltpu
from jax.experimental.pallas import tpu_sc as plsc

_N = 10000
_E = 320000
_DIN = 128
_HID = 128
_C = 64
_K = 10

_NCORE = 2
_NSUB = 16
_NW = _NCORE * _NSUB
_NPAD = 10240
_RPS = _NPAD // _NSUB
_CHUNK = 128
_SUP = 4
_CPW = 80
_NSUPW = _CPW // _SUP
_EPAD = _NW * _CPW * _CHUNK
_ECH = _EPAD // _CHUNK

_MESH = plsc.VectorSubcoreMesh(
    core_axis_name="c", subcore_axis_name="s",
    num_cores=_NCORE, num_subcores=_NSUB)

_I0 = np.int32(0)
_BM = 512
_GRID = _NPAD // _BM


def _mlp_body(x_ref, w1_ref, b1_ref, w2_ref, b2_ref, h_ref):
    xb = x_ref[...]
    h1 = lax.dot_general(xb, w1_ref[...], (((1,), (1,)), ((), ())),
                         preferred_element_type=jnp.float32)
    h1 = jnp.maximum(h1 + b1_ref[...], 0.0)
    h2 = lax.dot_general(h1, w2_ref[...], (((1,), (1,)), ((), ())),
                         preferred_element_type=jnp.float32)
    h_ref[...] = h2 + b2_ref[...]


def _mlp_call(xp, W1, b1, W2, b2):
    return pl.pallas_call(
        _mlp_body,
        grid=(_GRID,),
        in_specs=[
            pl.BlockSpec((_BM, _DIN), lambda i: (i, _I0)),
            pl.BlockSpec((_HID, _DIN), lambda i: (_I0, _I0)),
            pl.BlockSpec((1, _HID), lambda i: (_I0, _I0)),
            pl.BlockSpec((_C, _HID), lambda i: (_I0, _I0)),
            pl.BlockSpec((1, _C), lambda i: (_I0, _I0)),
        ],
        out_specs=pl.BlockSpec((_BM, _C), lambda i: (i, _I0)),
        out_shape=jax.ShapeDtypeStruct((_NPAD, _C), jnp.float32),
    )(xp, W1, b1.reshape(1, _HID), W2, b2.reshape(1, _C))


def _deg_body(colp_hbm, deg_out, deg_sh, idx_v, ones_v, zbuf_v, zdsem, dsem):
    c = lax.axis_index("c")
    s = lax.axis_index("s")
    w = c * _NSUB + s
    for i in range(8):
        ones_v[pl.ds(i * 16, 16)] = jnp.full((16,), 1.0, jnp.float32)
        zbuf_v[pl.ds(i * 16, 16)] = jnp.zeros((16,), jnp.float32)
    zd = [pltpu.async_copy(zbuf_v,
                           deg_sh.at[pl.ds(s * _RPS + i * 128, 128)], zdsem)
          for i in range(_RPS // 128)]
    pltpu.sync_copy(colp_hbm.at[pl.ds(w * _CPW, _CPW)], idx_v)
    for d in zd:
        d.wait()
    plsc.subcore_barrier()
    sd = [pltpu.async_copy(ones_v, deg_sh.at[idx_v.at[jnp.int32(j)]], dsem,
                           add=True)
          for j in range(_CPW)]
    for d in sd:
        d.wait()
    plsc.subcore_barrier()
    pltpu.sync_copy(deg_sh.at[pl.ds(s * _RPS, _RPS)],
                    deg_out.at[pl.ds(c * _NPAD + s * _RPS, _RPS)])


_deg_call = functools.partial(
    pl.kernel,
    out_type=jax.ShapeDtypeStruct((2 * _NPAD,), jnp.float32),
    mesh=_MESH,
    scratch_types=[
        pltpu.VMEM_SHARED((_NPAD,), jnp.float32),
        pltpu.VMEM((_CPW, _CHUNK), jnp.int32),
        pltpu.VMEM((_CHUNK,), jnp.float32),
        pltpu.VMEM((_CHUNK,), jnp.float32),
        pltpu.SemaphoreType.DMA,
        pltpu.SemaphoreType.DMA,
    ])(_deg_body)


def _prep_body(d0_ref, d1p_ref, h_ref, y0_ref, dinv1_ref, sq_ref):
    deg = d0_ref[...] + d1p_ref[...] + 1.0
    dinv = lax.rsqrt(deg)
    y0_ref[...] = h_ref[...] * dinv
    dinv1_ref[...] = dinv * dinv
    sq_ref[...] = deg * dinv


def _enc_body(x_ref, w1_ref, b1_ref, w2_ref, b2_ref, d0_ref, d1p_ref,
              h_ref, y0_ref, dinv1_ref, sq_ref):
    h1 = lax.dot_general(x_ref[...], w1_ref[...], (((1,), (1,)), ((), ())),
                         preferred_element_type=jnp.float32)
    h1 = jnp.maximum(h1 + b1_ref[...], 0.0)
    h2 = lax.dot_general(h1, w2_ref[...], (((1,), (1,)), ((), ())),
                         preferred_element_type=jnp.float32)
    h = h2 + b2_ref[...]
    h_ref[...] = h
    deg = d0_ref[...] + d1p_ref[...] + 1.0
    dinv = lax.rsqrt(deg)
    y0_ref[...] = h * dinv
    dinv1_ref[...] = dinv * dinv
    sq_ref[...] = deg * dinv


def _enc_call(xp, W1, b1, W2, b2, deg_p):
    return pl.pallas_call(
        _enc_body,
        grid=(_GRID,),
        in_specs=[
            pl.BlockSpec((_BM, _DIN), lambda i: (i, _I0)),
            pl.BlockSpec((_HID, _DIN), lambda i: (_I0, _I0)),
            pl.BlockSpec((1, _HID), lambda i: (_I0, _I0)),
            pl.BlockSpec((_C, _HID), lambda i: (_I0, _I0)),
            pl.BlockSpec((1, _C), lambda i: (_I0, _I0)),
            pl.BlockSpec((_BM, 1), lambda i: (i, _I0)),
            pl.BlockSpec((_BM, 1), lambda i: (i + _GRID, _I0)),
        ],
        out_specs=[
            pl.BlockSpec((_BM, _C), lambda i: (i, _I0)),
            pl.BlockSpec((_BM, _C), lambda i: (i, _I0)),
            pl.BlockSpec((_BM, 1), lambda i: (i, _I0)),
            pl.BlockSpec((_BM, 1), lambda i: (i, _I0)),
        ],
        out_shape=[
            jax.ShapeDtypeStruct((_NPAD, _C), jnp.float32),
            jax.ShapeDtypeStruct((_NPAD, _C), jnp.float32),
            jax.ShapeDtypeStruct((_NPAD, 1), jnp.float32),
            jax.ShapeDtypeStruct((_NPAD, 1), jnp.float32),
        ],
    )(xp, W1, b1.reshape(1, _HID), W2, b2.reshape(1, _C), deg_p, deg_p)


def _prop_body(y_hbm, rowp_hbm, colp_hbm, z_out, z_sh, idxr_v, idxc_v,
               val_v, zsem, gsem, ssem):
    c = lax.axis_index("c")
    s = lax.axis_index("s")
    w = c * _NSUB + s
    zinit = pltpu.async_copy(y_hbm.at[pl.ds(s * _RPS, _RPS)],
                             z_sh.at[pl.ds(s * _RPS, _RPS)], zsem)
    pltpu.sync_copy(rowp_hbm.at[pl.ds(w * _CPW, _CPW)], idxr_v)
    pltpu.sync_copy(colp_hbm.at[pl.ds(w * _CPW, _CPW)], idxc_v)

    def fire_gathers(j, b):
        return [pltpu.make_async_copy(
                    y_hbm.at[idxr_v.at[j * _SUP + jnp.int32(m)]],
                    val_v.at[b, jnp.int32(m)], gsem.at[b])
                for m in range(_SUP)]

    for d in fire_gathers(jnp.int32(0), jnp.int32(0)):
        d.start()
    for d in fire_gathers(jnp.int32(1), jnp.int32(1)):
        d.start()
    zinit.wait()
    plsc.subcore_barrier()

    @pl.loop(jnp.int32(0), jnp.int32(_NSUPW))
    def sup(j):
        b = lax.rem(j, jnp.int32(2))
        for d in fire_gathers(j, b):
            d.wait()
        sd = [pltpu.async_copy(val_v.at[b, jnp.int32(m)],
                               z_sh.at[idxc_v.at[j * _SUP + jnp.int32(m)]],
                               ssem, add=True)
              for m in range(_SUP)]
        for d in sd:
            d.wait()

        @pl.when(j < _NSUPW - 2)
        def _():
            for d in fire_gathers(j + 2, b):
                d.start()

    plsc.subcore_barrier()
    pltpu.sync_copy(z_sh.at[pl.ds(s * _RPS, _RPS)],
                    z_out.at[pl.ds(c * _NPAD + s * _RPS, _RPS)])


_prop_call = functools.partial(
    pl.kernel,
    out_type=jax.ShapeDtypeStruct((2 * _NPAD, _C), jnp.float32),
    mesh=_MESH,
    compiler_params=pltpu.CompilerParams(use_tc_tiling_on_sc=False),
    scratch_types=[
        pltpu.VMEM_SHARED((_NPAD, _C), jnp.float32),
        pltpu.VMEM((_CPW, _CHUNK), jnp.int32),
        pltpu.VMEM((_CPW, _CHUNK), jnp.int32),
        pltpu.VMEM((2, _SUP, _CHUNK, _C), jnp.float32),
        pltpu.SemaphoreType.DMA,
        pltpu.SemaphoreType.DMA((2,)),
        pltpu.SemaphoreType.DMA,
    ])(_prop_body)


def _ew_body(tk_ref, z0_ref, z1_ref, y_ref, d1_ref, hacc_ref,
             ynew_ref, hnew_ref):
    t = tk_ref[0, 0]
    sres = z0_ref[...] + z1_ref[...] - y_ref[...]
    u = sres * d1_ref[...]
    ynew_ref[...] = u
    hnew_ref[...] = hacc_ref[...] + t * u


def _ew_call(tk, zf, y, dinv1, hacc):
    return pl.pallas_call(
        _ew_body,
        grid=(_GRID,),
        in_specs=[
            pl.BlockSpec((1, 1), lambda i: (_I0, _I0)),
            pl.BlockSpec((_BM, _C), lambda i: (i, _I0)),
            pl.BlockSpec((_BM, _C), lambda i: (i + _GRID, _I0)),
            pl.BlockSpec((_BM, _C), lambda i: (i, _I0)),
            pl.BlockSpec((_BM, 1), lambda i: (i, _I0)),
            pl.BlockSpec((_BM, _C), lambda i: (i, _I0)),
        ],
        out_specs=[
            pl.BlockSpec((_BM, _C), lambda i: (i, _I0)),
            pl.BlockSpec((_BM, _C), lambda i: (i, _I0)),
        ],
        out_shape=[
            jax.ShapeDtypeStruct((_NPAD, _C), jnp.float32),
            jax.ShapeDtypeStruct((_NPAD, _C), jnp.float32),
        ],
    )(tk, zf, zf, y, dinv1, hacc)


def _final_body(t0_ref, h_ref, hacc_ref, sq_ref, out_ref):
    hidden = t0_ref[0, 0] * h_ref[...] + sq_ref[...] * hacc_ref[...]
    m = jnp.max(hidden, axis=1, keepdims=True)
    e = jnp.exp(hidden - m)
    lse = jnp.log(jnp.sum(e, axis=1, keepdims=True))
    out_ref[...] = hidden - m - lse


def _final_call(t0, h, hacc, sq):
    return pl.pallas_call(
        _final_body,
        grid=(_GRID,),
        in_specs=[
            pl.BlockSpec((1, 1), lambda i: (_I0, _I0)),
            pl.BlockSpec((_BM, _C), lambda i: (i, _I0)),
            pl.BlockSpec((_BM, _C), lambda i: (i, _I0)),
            pl.BlockSpec((_BM, 1), lambda i: (i, _I0)),
        ],
        out_specs=pl.BlockSpec((_BM, _C), lambda i: (i, _I0)),
        out_shape=jax.ShapeDtypeStruct((_NPAD, _C), jnp.float32),
    )(t0, h, hacc, sq)


def kernel(x, edge_index, W1, b1, W2, b2, temp):
    f32 = jnp.float32
    ei = edge_index.astype(jnp.int32)
    npad_e = _EPAD - _E
    pad_idx = _N + (jnp.arange(npad_e, dtype=jnp.int32) % (_NPAD - _N))
    rowp = jnp.concatenate([ei[0], pad_idx]).reshape(_ECH, _CHUNK)
    colp = jnp.concatenate([ei[1], pad_idx]).reshape(_ECH, _CHUNK)
    xp = jnp.pad(x.astype(f32), ((0, _NPAD - _N), (0, 0)))

    deg_p = _deg_call(colp).reshape(2 * _NPAD, 1)
    h, y, dinv1, sq = _enc_call(xp, W1.astype(f32), b1.astype(f32),
                                W2.astype(f32), b2.astype(f32), deg_p)

    temp = temp.astype(f32)
    hacc = jnp.zeros((_NPAD, _C), f32)
    for k in range(_K):
        zf = _prop_call(y, rowp, colp)
        y, hacc = _ew_call(temp[k + 1].reshape(1, 1), zf, y, dinv1, hacc)

    out = _final_call(temp[0].reshape(1, 1), h, hacc, sq)
    return out[:_N].astype(jnp.float64)


# speedup vs baseline: 508.6119x; 1.0003x over previous
import functools

import numpy as np

import jax
import jax.numpy as jnp
from jax import lax
from jax.experimental import pallas as pl
from jax.experimental.pallas import tpu as pltpu
from jax.experimental.pallas import tpu_sc as plsc

_N = 10000
_E = 320000
_DIN = 128
_HID = 128
_C = 64
_K = 10

_NCORE = 2
_NSUB = 16
_NW = _NCORE * _NSUB
_NPAD = 10240
_RPS = _NPAD // _NSUB
_CHUNK = 128
_SUP = 4
_CPW = 80
_NSUPW = _CPW // _SUP
_EPAD = _NW * _CPW * _CHUNK
_ECH = _EPAD // _CHUNK

_MESH = plsc.VectorSubcoreMesh(
    core_axis_name="c", subcore_axis_name="s",
    num_cores=_NCORE, num_subcores=_NSUB)

_I0 = np.int32(0)
_BM = 512
_GRID = _NPAD // _BM


def _deg_body(colp_hbm, deg_out, deg_sh, idx_v, ones_v, zbuf_v, zdsem, dsem):
    c = lax.axis_index("c")
    s = lax.axis_index("s")
    w = c * _NSUB + s
    for i in range(8):
        ones_v[pl.ds(i * 16, 16)] = jnp.full((16,), 1.0, jnp.float32)
        zbuf_v[pl.ds(i * 16, 16)] = jnp.zeros((16,), jnp.float32)
    zd = [pltpu.async_copy(zbuf_v,
                           deg_sh.at[pl.ds(s * _RPS + i * 128, 128)], zdsem)
          for i in range(_RPS // 128)]
    pltpu.sync_copy(colp_hbm.at[pl.ds(w * _CPW, _CPW)], idx_v)
    for d in zd:
        d.wait()
    plsc.subcore_barrier()
    sd = [pltpu.async_copy(ones_v, deg_sh.at[idx_v.at[jnp.int32(j)]], dsem,
                           add=True)
          for j in range(_CPW)]
    for d in sd:
        d.wait()
    plsc.subcore_barrier()
    pltpu.sync_copy(deg_sh.at[pl.ds(s * _RPS, _RPS)],
                    deg_out.at[pl.ds(c * _NPAD + s * _RPS, _RPS)])


_deg_call = functools.partial(
    pl.kernel,
    out_type=jax.ShapeDtypeStruct((2 * _NPAD,), jnp.float32),
    mesh=_MESH,
    scratch_types=[
        pltpu.VMEM_SHARED((_NPAD,), jnp.float32),
        pltpu.VMEM((_CPW, _CHUNK), jnp.int32),
        pltpu.VMEM((_CHUNK,), jnp.float32),
        pltpu.VMEM((_CHUNK,), jnp.float32),
        pltpu.SemaphoreType.DMA,
        pltpu.SemaphoreType.DMA,
    ])(_deg_body)


def _prep_body(d0_ref, d1p_ref, h_ref, y0_ref, dinv1_ref, sq_ref):
    deg = d0_ref[...] + d1p_ref[...] + 1.0
    dinv = lax.rsqrt(deg)
    y0_ref[...] = h_ref[...] * dinv
    dinv1_ref[...] = dinv * dinv
    sq_ref[...] = deg * dinv


def _enc_body(x_ref, w1_ref, b1_ref, w2_ref, b2_ref, d0_ref, d1p_ref,
              h_ref, y0_ref, dinv1_ref, sq_ref):
    h1 = lax.dot_general(x_ref[...], w1_ref[...], (((1,), (1,)), ((), ())),
                         preferred_element_type=jnp.float32)
    h1 = jnp.maximum(h1 + b1_ref[...], 0.0)
    h2 = lax.dot_general(h1, w2_ref[...], (((1,), (1,)), ((), ())),
                         preferred_element_type=jnp.float32)
    h = h2 + b2_ref[...]
    h_ref[...] = h
    deg = d0_ref[...] + d1p_ref[...] + 1.0
    dinv = lax.rsqrt(deg)
    y0_ref[...] = h * dinv
    dinv1_ref[...] = dinv * dinv
    sq_ref[...] = deg * dinv


def _enc_call(xp, W1, b1, W2, b2, deg_p):
    return pl.pallas_call(
        _enc_body,
        grid=(_GRID,),
        in_specs=[
            pl.BlockSpec((_BM, _DIN), lambda i: (i, _I0)),
            pl.BlockSpec((_HID, _DIN), lambda i: (_I0, _I0)),
            pl.BlockSpec((1, _HID), lambda i: (_I0, _I0)),
            pl.BlockSpec((_C, _HID), lambda i: (_I0, _I0)),
            pl.BlockSpec((1, _C), lambda i: (_I0, _I0)),
            pl.BlockSpec((_BM, 1), lambda i: (i, _I0)),
            pl.BlockSpec((_BM, 1), lambda i: (i + _GRID, _I0)),
        ],
        out_specs=[
            pl.BlockSpec((_BM, _C), lambda i: (i, _I0)),
            pl.BlockSpec((_BM, _C), lambda i: (i, _I0)),
            pl.BlockSpec((_BM, 1), lambda i: (i, _I0)),
            pl.BlockSpec((_BM, 1), lambda i: (i, _I0)),
        ],
        out_shape=[
            jax.ShapeDtypeStruct((_NPAD, _C), jnp.float32),
            jax.ShapeDtypeStruct((_NPAD, _C), jnp.float32),
            jax.ShapeDtypeStruct((_NPAD, 1), jnp.float32),
            jax.ShapeDtypeStruct((_NPAD, 1), jnp.float32),
        ],
    )(xp, W1, b1.reshape(1, _HID), W2, b2.reshape(1, _C), deg_p, deg_p)


def _prop_body(y_hbm, rowp_hbm, colp_hbm, z_out, z_sh, idxr_v, idxc_v,
               val_v, zsem, gsem, ssem):
    c = lax.axis_index("c")
    s = lax.axis_index("s")
    w = c * _NSUB + s
    zinit = pltpu.async_copy(y_hbm.at[pl.ds(s * _RPS, _RPS)],
                             z_sh.at[pl.ds(s * _RPS, _RPS)], zsem)
    pltpu.sync_copy(rowp_hbm.at[pl.ds(w * _CPW, _CPW)], idxr_v)
    pltpu.sync_copy(colp_hbm.at[pl.ds(w * _CPW, _CPW)], idxc_v)

    def fire_gathers(j, b):
        return [pltpu.make_async_copy(
                    y_hbm.at[idxr_v.at[j * _SUP + jnp.int32(m)]],
                    val_v.at[b, jnp.int32(m)], gsem.at[b])
                for m in range(_SUP)]

    for d in fire_gathers(jnp.int32(0), jnp.int32(0)):
        d.start()
    for d in fire_gathers(jnp.int32(1), jnp.int32(1)):
        d.start()
    zinit.wait()
    plsc.subcore_barrier()

    @pl.loop(jnp.int32(0), jnp.int32(_NSUPW))
    def sup(j):
        b = lax.rem(j, jnp.int32(2))
        for d in fire_gathers(j, b):
            d.wait()
        sd = [pltpu.async_copy(val_v.at[b, jnp.int32(m)],
                               z_sh.at[idxc_v.at[j * _SUP + jnp.int32(m)]],
                               ssem, add=True)
              for m in range(_SUP)]
        for d in sd:
            d.wait()

        @pl.when(j < _NSUPW - 2)
        def _():
            for d in fire_gathers(j + 2, b):
                d.start()

    plsc.subcore_barrier()
    pltpu.sync_copy(z_sh.at[pl.ds(s * _RPS, _RPS)],
                    z_out.at[pl.ds(c * _NPAD + s * _RPS, _RPS)])


_prop_call = functools.partial(
    pl.kernel,
    out_type=jax.ShapeDtypeStruct((2 * _NPAD, _C), jnp.float32),
    mesh=_MESH,
    compiler_params=pltpu.CompilerParams(use_tc_tiling_on_sc=False),
    scratch_types=[
        pltpu.VMEM_SHARED((_NPAD, _C), jnp.float32),
        pltpu.VMEM((_CPW, _CHUNK), jnp.int32),
        pltpu.VMEM((_CPW, _CHUNK), jnp.int32),
        pltpu.VMEM((2, _SUP, _CHUNK, _C), jnp.float32),
        pltpu.SemaphoreType.DMA,
        pltpu.SemaphoreType.DMA((2,)),
        pltpu.SemaphoreType.DMA,
    ])(_prop_body)


def _ew_body(tk_ref, z0_ref, z1_ref, y_ref, d1_ref, hacc_ref,
             ynew_ref, hnew_ref):
    t = tk_ref[0, 0]
    sres = z0_ref[...] + z1_ref[...] - y_ref[...]
    u = sres * d1_ref[...]
    ynew_ref[...] = u
    hnew_ref[...] = hacc_ref[...] + t * u


def _ew_call(tk, zf, y, dinv1, hacc):
    return pl.pallas_call(
        _ew_body,
        grid=(_GRID,),
        in_specs=[
            pl.BlockSpec((1, 1), lambda i: (_I0, _I0)),
            pl.BlockSpec((_BM, _C), lambda i: (i, _I0)),
            pl.BlockSpec((_BM, _C), lambda i: (i + _GRID, _I0)),
            pl.BlockSpec((_BM, _C), lambda i: (i, _I0)),
            pl.BlockSpec((_BM, 1), lambda i: (i, _I0)),
            pl.BlockSpec((_BM, _C), lambda i: (i, _I0)),
        ],
        out_specs=[
            pl.BlockSpec((_BM, _C), lambda i: (i, _I0)),
            pl.BlockSpec((_BM, _C), lambda i: (i, _I0)),
        ],
        out_shape=[
            jax.ShapeDtypeStruct((_NPAD, _C), jnp.float32),
            jax.ShapeDtypeStruct((_NPAD, _C), jnp.float32),
        ],
    )(tk, zf, zf, y, dinv1, hacc)


def _final_body(t0_ref, h_ref, hacc_ref, sq_ref, out_ref):
    hidden = t0_ref[0, 0] * h_ref[...] + sq_ref[...] * hacc_ref[...]
    m = jnp.max(hidden, axis=1, keepdims=True)
    e = jnp.exp(hidden - m)
    lse = jnp.log(jnp.sum(e, axis=1, keepdims=True))
    out_ref[...] = hidden - m - lse


def _final_call(t0, h, hacc, sq):
    return pl.pallas_call(
        _final_body,
        grid=(_GRID,),
        in_specs=[
            pl.BlockSpec((1, 1), lambda i: (_I0, _I0)),
            pl.BlockSpec((_BM, _C), lambda i: (i, _I0)),
            pl.BlockSpec((_BM, _C), lambda i: (i, _I0)),
            pl.BlockSpec((_BM, 1), lambda i: (i, _I0)),
        ],
        out_specs=pl.BlockSpec((_BM, _C), lambda i: (i, _I0)),
        out_shape=jax.ShapeDtypeStruct((_NPAD, _C), jnp.float32),
    )(t0, h, hacc, sq)


def kernel(x, edge_index, W1, b1, W2, b2, temp):
    f32 = jnp.float32
    ei = edge_index.astype(jnp.int32)
    npad_e = _EPAD - _E
    pad_idx = _N + (jnp.arange(npad_e, dtype=jnp.int32) % (_NPAD - _N))
    rowp = jnp.concatenate([ei[0], pad_idx]).reshape(_ECH, _CHUNK)
    colp = jnp.concatenate([ei[1], pad_idx]).reshape(_ECH, _CHUNK)
    xp = jnp.pad(x.astype(f32), ((0, _NPAD - _N), (0, 0)))

    deg_p = _deg_call(colp).reshape(2 * _NPAD, 1)
    h, y, dinv1, sq = _enc_call(xp, W1.astype(f32), b1.astype(f32),
                                W2.astype(f32), b2.astype(f32), deg_p)

    temp = temp.astype(f32)
    hacc = jnp.zeros((_NPAD, _C), f32)
    for k in range(_K):
        zf = _prop_call(y, rowp, colp)
        y, hacc = _ew_call(temp[k + 1].reshape(1, 1), zf, y, dinv1, hacc)

    out = _final_call(temp[0].reshape(1, 1), h, hacc, sq)
    return out[:_N].astype(jnp.float64)
